# Initial kernel scaffold; baseline (speedup 1.0000x reference)
import jax
import jax.numpy as jnp
from jax.experimental import pallas as pl


def kernel(x, edge_index, edge_weight, W1a, W1s, b1, W2a, W2s, b2, Wd, bd, Wx1, bx1, Wx2, bx2, Wx3, bx3, Wpi, bpi):
    raise NotImplementedError("write your pallas kernel here")


# speedup vs baseline: 1.0947x; 1.0947x over previous
import functools

import jax
import jax.numpy as jnp
from jax.experimental import pallas as pl
from jax.experimental.pallas import tpu as pltpu

N = 10000
E = 320000
IN = 128
H = 128
L = 15
ADJ = 32

ROW_BLK = 1000
BIL_I = 400


def _mm_body(x_ref, w_ref, o_ref):
    o_ref[...] = jnp.dot(x_ref[...], w_ref[...],
                         preferred_element_type=jnp.float32)


def _matmul(x, w):
    m, k = x.shape
    k2, n = w.shape
    return pl.pallas_call(
        _mm_body,
        grid=(m // ROW_BLK,),
        in_specs=[
            pl.BlockSpec((ROW_BLK, k), lambda i: (i, 0)),
            pl.BlockSpec((k, n), lambda i: (0, 0)),
        ],
        out_specs=pl.BlockSpec((ROW_BLK, n), lambda i: (i, 0)),
        out_shape=jax.ShapeDtypeStruct((m, n), jnp.float32),
    )(x, w)


def _stage2_body(s1_ref, x_ref, w1s_ref, b1_ref, w2a_ref, w2s_ref, b2_ref,
                 t2_ref, u2_ref):
    h = jax.nn.relu(s1_ref[...] + jnp.dot(x_ref[...], w1s_ref[...],
                                          preferred_element_type=jnp.float32)
                    + b1_ref[...])
    t2_ref[...] = jnp.dot(h, w2a_ref[...], preferred_element_type=jnp.float32)
    u2_ref[...] = jnp.dot(h, w2s_ref[...],
                          preferred_element_type=jnp.float32) + b2_ref[...]


def _stage2(s1, x, W1s, b1, W2a, W2s, b2):
    return pl.pallas_call(
        _stage2_body,
        grid=(N // ROW_BLK,),
        in_specs=[
            pl.BlockSpec((ROW_BLK, H), lambda i: (i, 0)),
            pl.BlockSpec((ROW_BLK, IN), lambda i: (i, 0)),
            pl.BlockSpec((IN, H), lambda i: (0, 0)),
            pl.BlockSpec((1, H), lambda i: (0, 0)),
            pl.BlockSpec((H, L), lambda i: (0, 0)),
            pl.BlockSpec((H, L), lambda i: (0, 0)),
            pl.BlockSpec((1, L), lambda i: (0, 0)),
        ],
        out_specs=[
            pl.BlockSpec((ROW_BLK, L), lambda i: (i, 0)),
            pl.BlockSpec((ROW_BLK, L), lambda i: (i, 0)),
        ],
        out_shape=[
            jax.ShapeDtypeStruct((N, L), jnp.float32),
            jax.ShapeDtypeStruct((N, L), jnp.float32),
        ],
    )(s1, x, W1s, b1.reshape(1, H), W2a, W2s, b2.reshape(1, L))


def _decoder_body(s2_ref, u2_ref, wd_ref, bd_ref, wx1_ref, bx1_ref,
                  wx2_ref, bx2_ref, wx3_ref, bx3_ref, wpi_ref, bpi_ref,
                  ha_ref, p_ref):
    z = s2_ref[...] + u2_ref[...]
    ha_ref[...] = jnp.dot(z, wd_ref[...],
                          preferred_element_type=jnp.float32) + bd_ref[...]
    d = jax.nn.relu(jnp.dot(z, wx1_ref[...],
                            preferred_element_type=jnp.float32) + bx1_ref[...])
    d = jax.nn.relu(jnp.dot(d, wx2_ref[...],
                            preferred_element_type=jnp.float32) + bx2_ref[...])
    d = jax.nn.relu(jnp.dot(d, wx3_ref[...],
                            preferred_element_type=jnp.float32) + bx3_ref[...])
    logits = jnp.dot(d, wpi_ref[...],
                     preferred_element_type=jnp.float32) + bpi_ref[...]
    m = jnp.max(logits, axis=-1, keepdims=True)
    e = jnp.exp(logits - m)
    p_ref[...] = e / jnp.sum(e, axis=-1, keepdims=True)


def _decoder(s2, u2, Wd, bd, Wx1, bx1, Wx2, bx2, Wx3, bx3, Wpi, bpi):
    d0, d1, d2 = Wx1.shape[1], Wx2.shape[1], Wx3.shape[1]
    full = lambda shape: pl.BlockSpec(shape, lambda i: tuple(0 for _ in shape))
    return pl.pallas_call(
        _decoder_body,
        grid=(N // ROW_BLK,),
        in_specs=[
            pl.BlockSpec((ROW_BLK, L), lambda i: (i, 0)),
            pl.BlockSpec((ROW_BLK, L), lambda i: (i, 0)),
            full((L, ADJ)), full((1, ADJ)),
            full((L, d0)), full((1, d0)),
            full((d0, d1)), full((1, d1)),
            full((d1, d2)), full((1, d2)),
            full((d2, IN)), full((1, IN)),
        ],
        out_specs=[
            pl.BlockSpec((ROW_BLK, ADJ), lambda i: (i, 0)),
            pl.BlockSpec((ROW_BLK, IN), lambda i: (i, 0)),
        ],
        out_shape=[
            jax.ShapeDtypeStruct((N, ADJ), jnp.float32),
            jax.ShapeDtypeStruct((N, IN), jnp.float32),
        ],
    )(s2, u2, Wd, bd.reshape(1, ADJ), Wx1, bx1.reshape(1, d0),
      Wx2, bx2.reshape(1, d1), Wx3, bx3.reshape(1, d2),
      Wpi, bpi.reshape(1, IN))


def _bilinear_body(hi_ref, hj_ref, o_ref):
    prod = jax.lax.dot_general(
        hi_ref[...], hj_ref[...],
        dimension_numbers=(((1,), (1,)), ((), ())),
        preferred_element_type=jnp.float32)
    o_ref[...] = jax.nn.sigmoid(prod)


def _bilinear(hA):
    return pl.pallas_call(
        _bilinear_body,
        grid=(N // BIL_I,),
        in_specs=[
            pl.BlockSpec((BIL_I, ADJ), lambda i: (i, 0)),
            pl.BlockSpec((N, ADJ), lambda i: (0, 0)),
        ],
        out_specs=pl.BlockSpec((BIL_I, N), lambda i: (i, 0)),
        out_shape=jax.ShapeDtypeStruct((N, N), jnp.float32),
    )(hA, hA)


def _spmm(edge_index, edge_weight, h):
    src = edge_index[0]
    dst = edge_index[1]
    msgs = jnp.take(h, src, axis=0) * edge_weight[:, None]
    return jax.ops.segment_sum(msgs, dst, num_segments=N)


def kernel(x, edge_index, edge_weight, W1a, W1s, b1, W2a, W2s, b2,
           Wd, bd, Wx1, bx1, Wx2, bx2, Wx3, bx3, Wpi, bpi):
    t1 = _matmul(x, W1a)
    s1 = _spmm(edge_index, edge_weight, t1)
    t2, u2 = _stage2(s1, x, W1s, b1, W2a, W2s, b2)
    s2 = _spmm(edge_index, edge_weight, t2)
    hA, P = _decoder(s2, u2, Wd, bd, Wx1, bx1, Wx2, bx2, Wx3, bx3, Wpi, bpi)
    A_out = _bilinear(hA)
    return (A_out, P)


# speedup vs baseline: 6.2167x; 5.6791x over previous
import functools

import jax
import jax.numpy as jnp
from jax import lax
from jax.experimental import pallas as pl
from jax.experimental.pallas import tpu as pltpu
from jax.experimental.pallas import tpu_sc as plsc

N = 10000
E = 320000
IN = 128
H = 128
L = 15
LP = 16
ADJ = 32

ROW_BLK = 1000
BIL_I = 400

NW = 32
EPW = E // NW
CHUNK = 80
NCH = EPW // CHUNK
NG = 5
KJ = NCH // NG
RSTRIDE = 624
REXT = 640


def _mm_body(x_ref, w_ref, o_ref):
    o_ref[...] = jnp.dot(x_ref[...], w_ref[...],
                         preferred_element_type=jnp.float32)


def _matmul(x, w):
    m, k = x.shape
    k2, n = w.shape
    return pl.pallas_call(
        _mm_body,
        grid=(m // ROW_BLK,),
        in_specs=[
            pl.BlockSpec((ROW_BLK, k), lambda i: (i, 0)),
            pl.BlockSpec((k, n), lambda i: (0, 0)),
        ],
        out_specs=pl.BlockSpec((ROW_BLK, n), lambda i: (i, 0)),
        out_shape=jax.ShapeDtypeStruct((m, n), jnp.float32),
    )(x, w)


def _stage2_body(s1_ref, x_ref, w1s_ref, b1_ref, w2a_ref, w2s_ref, b2_ref,
                 t2_ref, u2_ref):
    s1 = s1_ref[0] + s1_ref[1]
    h = jax.nn.relu(s1 + jnp.dot(x_ref[...], w1s_ref[...],
                                 preferred_element_type=jnp.float32)
                    + b1_ref[...])
    t2_ref[...] = jnp.dot(h, w2a_ref[...], preferred_element_type=jnp.float32)
    u2_ref[...] = jnp.dot(h, w2s_ref[...],
                          preferred_element_type=jnp.float32) + b2_ref[...]


def _stage2(s1p, x, W1s, b1, W2a, W2s, b2):
    return pl.pallas_call(
        _stage2_body,
        grid=(N // ROW_BLK,),
        in_specs=[
            pl.BlockSpec((2, ROW_BLK, H), lambda i: (0, i, 0)),
            pl.BlockSpec((ROW_BLK, IN), lambda i: (i, 0)),
            pl.BlockSpec((IN, H), lambda i: (0, 0)),
            pl.BlockSpec((1, H), lambda i: (0, 0)),
            pl.BlockSpec((H, LP), lambda i: (0, 0)),
            pl.BlockSpec((H, LP), lambda i: (0, 0)),
            pl.BlockSpec((1, LP), lambda i: (0, 0)),
        ],
        out_specs=[
            pl.BlockSpec((ROW_BLK, LP), lambda i: (i, 0)),
            pl.BlockSpec((ROW_BLK, LP), lambda i: (i, 0)),
        ],
        out_shape=[
            jax.ShapeDtypeStruct((N, LP), jnp.float32),
            jax.ShapeDtypeStruct((N, LP), jnp.float32),
        ],
    )(s1p, x, W1s, b1.reshape(1, H), W2a, W2s, b2.reshape(1, LP))


def _decoder_body(s2_ref, u2_ref, wd_ref, bd_ref, wx1_ref, bx1_ref,
                  wx2_ref, bx2_ref, wx3_ref, bx3_ref, wpi_ref, bpi_ref,
                  ha_ref, p_ref):
    z = s2_ref[0] + s2_ref[1] + u2_ref[...]
    ha_ref[...] = jnp.dot(z, wd_ref[...],
                          preferred_element_type=jnp.float32) + bd_ref[...]
    d = jax.nn.relu(jnp.dot(z, wx1_ref[...],
                            preferred_element_type=jnp.float32) + bx1_ref[...])
    d = jax.nn.relu(jnp.dot(d, wx2_ref[...],
                            preferred_element_type=jnp.float32) + bx2_ref[...])
    d = jax.nn.relu(jnp.dot(d, wx3_ref[...],
                            preferred_element_type=jnp.float32) + bx3_ref[...])
    logits = jnp.dot(d, wpi_ref[...],
                     preferred_element_type=jnp.float32) + bpi_ref[...]
    m = jnp.max(logits, axis=-1, keepdims=True)
    e = jnp.exp(logits - m)
    p_ref[...] = e / jnp.sum(e, axis=-1, keepdims=True)


def _decoder(s2p, u2, Wd, bd, Wx1, bx1, Wx2, bx2, Wx3, bx3, Wpi, bpi):
    d0, d1, d2 = Wx1.shape[1], Wx2.shape[1], Wx3.shape[1]
    full = lambda shape: pl.BlockSpec(shape, lambda i: tuple(0 for _ in shape))
    return pl.pallas_call(
        _decoder_body,
        grid=(N // ROW_BLK,),
        in_specs=[
            pl.BlockSpec((2, ROW_BLK, LP), lambda i: (0, i, 0)),
            pl.BlockSpec((ROW_BLK, LP), lambda i: (i, 0)),
            full((LP, ADJ)), full((1, ADJ)),
            full((LP, d0)), full((1, d0)),
            full((d0, d1)), full((1, d1)),
            full((d1, d2)), full((1, d2)),
            full((d2, IN)), full((1, IN)),
        ],
        out_specs=[
            pl.BlockSpec((ROW_BLK, ADJ), lambda i: (i, 0)),
            pl.BlockSpec((ROW_BLK, IN), lambda i: (i, 0)),
        ],
        out_shape=[
            jax.ShapeDtypeStruct((N, ADJ), jnp.float32),
            jax.ShapeDtypeStruct((N, IN), jnp.float32),
        ],
    )(s2p, u2, Wd, bd.reshape(1, ADJ), Wx1, bx1.reshape(1, d0),
      Wx2, bx2.reshape(1, d1), Wx3, bx3.reshape(1, d2),
      Wpi, bpi.reshape(1, IN))


def _bilinear_body(hi_ref, hj_ref, o_ref):
    prod = jax.lax.dot_general(
        hi_ref[...], hj_ref[...],
        dimension_numbers=(((1,), (1,)), ((), ())),
        preferred_element_type=jnp.float32)
    o_ref[...] = jax.nn.sigmoid(prod)


def _bilinear(hA):
    return pl.pallas_call(
        _bilinear_body,
        grid=(N // BIL_I,),
        in_specs=[
            pl.BlockSpec((BIL_I, ADJ), lambda i: (i, 0)),
            pl.BlockSpec((N, ADJ), lambda i: (0, 0)),
        ],
        out_specs=pl.BlockSpec((BIL_I, N), lambda i: (i, 0)),
        out_shape=jax.ShapeDtypeStruct((N, N), jnp.float32),
    )(hA, hA)


@functools.lru_cache(maxsize=None)
def _make_sc_spmm(D):
    mesh = plsc.VectorSubcoreMesh(core_axis_name="c", subcore_axis_name="s")

    def body(y_hbm, src_hbm, dst_hbm, w_hbm, out_hbm,
             acc_sh, src_v, dst_v, w_v, rows_v, sem):
        c = lax.axis_index("c")
        s = lax.axis_index("s")
        g = c * 16 + s

        def _zrow(e, _):
            for dd in range(D // 16):
                rows_v[e, pl.ds(dd * 16, 16)] = jnp.zeros((16,), jnp.float32)
            return 0
        lax.fori_loop(0, CHUNK, _zrow, 0)
        r0 = s * RSTRIDE
        for k in range(REXT // CHUNK):
            pltpu.sync_copy(rows_v, acc_sh.at[pl.ds(r0 + k * CHUNK, CHUNK)])

        plsc.subcore_barrier()

        def group_body(grp, _):
            gg = g * NG + grp
            pltpu.sync_copy(src_hbm.at[gg], src_v)
            pltpu.sync_copy(dst_hbm.at[gg], dst_v)
            pltpu.sync_copy(w_hbm.at[gg], w_v)

            def chunk_body(j, _):
                pltpu.async_copy(y_hbm.at[src_v.at[j]], rows_v, sem).wait()
                for b in range(CHUNK // 16):
                    wv = w_v[j, pl.ds(b * 16, 16)]
                    for l in range(16):
                        ws = jnp.full((16,), wv[l], dtype=jnp.float32)
                        e = b * 16 + l
                        for dd in range(D // 16):
                            sl = pl.ds(dd * 16, 16)
                            rows_v[e, sl] = rows_v[e, sl] * ws
                pltpu.sync_copy(rows_v, acc_sh.at[dst_v.at[j]], add=True)
                return 0
            lax.fori_loop(0, KJ, chunk_body, 0)
            return 0
        lax.fori_loop(0, NG, group_body, 0)

        plsc.subcore_barrier()
        pltpu.sync_copy(acc_sh.at[pl.ds(r0, REXT)],
                        out_hbm.at[c, pl.ds(r0, REXT)])

    return pl.kernel(
        body,
        out_type=jax.ShapeDtypeStruct((2, N, D), jnp.float32),
        mesh=mesh,
        compiler_params=pltpu.CompilerParams(use_tc_tiling_on_sc=False),
        scratch_types=[
            pltpu.VMEM_SHARED((N, D), jnp.float32),
            pltpu.VMEM((KJ, CHUNK), jnp.int32),
            pltpu.VMEM((KJ, CHUNK), jnp.int32),
            pltpu.VMEM((KJ, CHUNK), jnp.float32),
            pltpu.VMEM((CHUNK, D), jnp.float32),
            pltpu.SemaphoreType.DMA,
        ],
    )


def kernel(x, edge_index, edge_weight, W1a, W1s, b1, W2a, W2s, b2,
           Wd, bd, Wx1, bx1, Wx2, bx2, Wx3, bx3, Wpi, bpi):
    srcg = edge_index[0].reshape(NW * NG, KJ, CHUNK)
    dstg = edge_index[1].reshape(NW * NG, KJ, CHUNK)
    wg = edge_weight.reshape(NW * NG, KJ, CHUNK)
    W2a_p = jnp.pad(W2a, ((0, 0), (0, LP - L)))
    W2s_p = jnp.pad(W2s, ((0, 0), (0, LP - L)))
    b2_p = jnp.pad(b2, (0, LP - L))
    Wd_p = jnp.pad(Wd, ((0, LP - L), (0, 0)))
    Wx1_p = jnp.pad(Wx1, ((0, LP - L), (0, 0)))

    t1 = _matmul(x, W1a)
    s1p = _make_sc_spmm(IN)(t1, srcg, dstg, wg)
    t2, u2 = _stage2(s1p, x, W1s, b1, W2a_p, W2s_p, b2_p)
    s2p = _make_sc_spmm(LP)(t2, srcg, dstg, wg)
    hA, P = _decoder(s2p, u2, Wd_p, bd, Wx1_p, bx1, Wx2, bx2, Wx3, bx3,
                     Wpi, bpi)
    A_out = _bilinear(hA)
    return (A_out, P)


# speedup vs baseline: 8.4663x; 1.3619x over previous
import functools

import jax
import jax.numpy as jnp
from jax import lax
from jax.experimental import pallas as pl
from jax.experimental.pallas import tpu as pltpu
from jax.experimental.pallas import tpu_sc as plsc

N = 10000
E = 320000
IN = 128
H = 128
L = 15
LP = 16
ADJ = 32

ROW_BLK = 1000
BIL_I = 400

NW = 32
EPW = E // NW
CHUNK = 80
NCH = EPW // CHUNK
NG = 5
KJ = NCH // NG
RSTRIDE = 624
REXT = 640


def _mm_body(x_ref, w_ref, o_ref):
    o_ref[...] = jnp.dot(x_ref[...], w_ref[...],
                         preferred_element_type=jnp.float32)


def _matmul(x, w):
    m, k = x.shape
    k2, n = w.shape
    return pl.pallas_call(
        _mm_body,
        grid=(m // ROW_BLK,),
        in_specs=[
            pl.BlockSpec((ROW_BLK, k), lambda i: (i, 0)),
            pl.BlockSpec((k, n), lambda i: (0, 0)),
        ],
        out_specs=pl.BlockSpec((ROW_BLK, n), lambda i: (i, 0)),
        out_shape=jax.ShapeDtypeStruct((m, n), jnp.float32),
    )(x, w)


def _stage2_body(s1_ref, x_ref, w1s_ref, b1_ref, w2a_ref, w2s_ref, b2_ref,
                 t2_ref, u2_ref):
    s1 = s1_ref[0] + s1_ref[1]
    h = jax.nn.relu(s1 + jnp.dot(x_ref[...], w1s_ref[...],
                                 preferred_element_type=jnp.float32)
                    + b1_ref[...])
    t2_ref[...] = jnp.dot(h, w2a_ref[...], preferred_element_type=jnp.float32)
    u2_ref[...] = jnp.dot(h, w2s_ref[...],
                          preferred_element_type=jnp.float32) + b2_ref[...]


def _stage2(s1p, x, W1s, b1, W2a, W2s, b2):
    return pl.pallas_call(
        _stage2_body,
        grid=(N // ROW_BLK,),
        in_specs=[
            pl.BlockSpec((2, ROW_BLK, H), lambda i: (0, i, 0)),
            pl.BlockSpec((ROW_BLK, IN), lambda i: (i, 0)),
            pl.BlockSpec((IN, H), lambda i: (0, 0)),
            pl.BlockSpec((1, H), lambda i: (0, 0)),
            pl.BlockSpec((H, LP), lambda i: (0, 0)),
            pl.BlockSpec((H, LP), lambda i: (0, 0)),
            pl.BlockSpec((1, LP), lambda i: (0, 0)),
        ],
        out_specs=[
            pl.BlockSpec((ROW_BLK, LP), lambda i: (i, 0)),
            pl.BlockSpec((ROW_BLK, LP), lambda i: (i, 0)),
        ],
        out_shape=[
            jax.ShapeDtypeStruct((N, LP), jnp.float32),
            jax.ShapeDtypeStruct((N, LP), jnp.float32),
        ],
    )(s1p, x, W1s, b1.reshape(1, H), W2a, W2s, b2.reshape(1, LP))


def _decoder_body(s2_ref, u2_ref, wd_ref, bd_ref, wx1_ref, bx1_ref,
                  wx2_ref, bx2_ref, wx3_ref, bx3_ref, wpi_ref, bpi_ref,
                  ha_ref, p_ref):
    z = s2_ref[0] + s2_ref[1] + u2_ref[...]
    ha_ref[...] = jnp.dot(z, wd_ref[...],
                          preferred_element_type=jnp.float32) + bd_ref[...]
    d = jax.nn.relu(jnp.dot(z, wx1_ref[...],
                            preferred_element_type=jnp.float32) + bx1_ref[...])
    d = jax.nn.relu(jnp.dot(d, wx2_ref[...],
                            preferred_element_type=jnp.float32) + bx2_ref[...])
    d = jax.nn.relu(jnp.dot(d, wx3_ref[...],
                            preferred_element_type=jnp.float32) + bx3_ref[...])
    logits = jnp.dot(d, wpi_ref[...],
                     preferred_element_type=jnp.float32) + bpi_ref[...]
    m = jnp.max(logits, axis=-1, keepdims=True)
    e = jnp.exp(logits - m)
    p_ref[...] = e / jnp.sum(e, axis=-1, keepdims=True)


def _decoder(s2p, u2, Wd, bd, Wx1, bx1, Wx2, bx2, Wx3, bx3, Wpi, bpi):
    d0, d1, d2 = Wx1.shape[1], Wx2.shape[1], Wx3.shape[1]
    full = lambda shape: pl.BlockSpec(shape, lambda i: tuple(0 for _ in shape))
    return pl.pallas_call(
        _decoder_body,
        grid=(N // ROW_BLK,),
        in_specs=[
            pl.BlockSpec((2, ROW_BLK, LP), lambda i: (0, i, 0)),
            pl.BlockSpec((ROW_BLK, LP), lambda i: (i, 0)),
            full((LP, ADJ)), full((1, ADJ)),
            full((LP, d0)), full((1, d0)),
            full((d0, d1)), full((1, d1)),
            full((d1, d2)), full((1, d2)),
            full((d2, IN)), full((1, IN)),
        ],
        out_specs=[
            pl.BlockSpec((ROW_BLK, ADJ), lambda i: (i, 0)),
            pl.BlockSpec((ROW_BLK, IN), lambda i: (i, 0)),
        ],
        out_shape=[
            jax.ShapeDtypeStruct((N, ADJ), jnp.float32),
            jax.ShapeDtypeStruct((N, IN), jnp.float32),
        ],
    )(s2p, u2, Wd, bd.reshape(1, ADJ), Wx1, bx1.reshape(1, d0),
      Wx2, bx2.reshape(1, d1), Wx3, bx3.reshape(1, d2),
      Wpi, bpi.reshape(1, IN))


def _bilinear_body(hi_ref, hj_ref, o_ref):
    prod = jax.lax.dot_general(
        hi_ref[...], hj_ref[...],
        dimension_numbers=(((1,), (1,)), ((), ())),
        preferred_element_type=jnp.float32)
    o_ref[...] = jax.nn.sigmoid(prod)


def _bilinear(hA):
    return pl.pallas_call(
        _bilinear_body,
        grid=(N // BIL_I,),
        in_specs=[
            pl.BlockSpec((BIL_I, ADJ), lambda i: (i, 0)),
            pl.BlockSpec((N, ADJ), lambda i: (0, 0)),
        ],
        out_specs=pl.BlockSpec((BIL_I, N), lambda i: (i, 0)),
        out_shape=jax.ShapeDtypeStruct((N, N), jnp.float32),
    )(hA, hA)


@functools.lru_cache(maxsize=None)
def _make_sc_spmm(D):
    mesh = plsc.VectorSubcoreMesh(core_axis_name="c", subcore_axis_name="s")

    def body(y_hbm, src_hbm, dst_hbm, w_hbm, out_hbm,
             acc_sh, src_v, dst_v, w_v, rows_a, rows_b, sem_a, sem_b):
        c = lax.axis_index("c")
        s = lax.axis_index("s")
        g = c * 16 + s

        def _zrow(e, _):
            for dd in range(D // 16):
                rows_a[e, pl.ds(dd * 16, 16)] = jnp.zeros((16,), jnp.float32)
            return 0
        lax.fori_loop(0, CHUNK, _zrow, 0)
        r0 = s * RSTRIDE
        for k in range(REXT // CHUNK):
            pltpu.sync_copy(rows_a, acc_sh.at[pl.ds(r0 + k * CHUNK, CHUNK)])

        plsc.subcore_barrier()

        def _gather(j, rows, sem):
            return pltpu.make_async_copy(y_hbm.at[src_v.at[j]], rows, sem)

        def _scale(rows, j):
            def _sb(b, _):
                wv = w_v[j, pl.ds(b * 16, 16)]
                for l in range(16):
                    ws = jnp.full((16,), wv[l], dtype=jnp.float32)
                    for dd in range(D // 16):
                        sl = pl.ds(dd * 16, 16)
                        rows[b * 16 + l, sl] = rows[b * 16 + l, sl] * ws
                return 0
            lax.fori_loop(0, CHUNK // 16, _sb, 0)

        def group_body(grp, _):
            gg = g * NG + grp
            pltpu.sync_copy(src_hbm.at[gg], src_v)
            pltpu.sync_copy(dst_hbm.at[gg], dst_v)
            pltpu.sync_copy(w_hbm.at[gg], w_v)

            _gather(0, rows_a, sem_a).start()

            def pair_body(p, _):
                ja = 2 * p
                _gather(ja + 1, rows_b, sem_b).start()
                _gather(ja, rows_a, sem_a).wait()
                _scale(rows_a, ja)
                pltpu.sync_copy(rows_a, acc_sh.at[dst_v.at[ja]], add=True)
                _gather(ja + 2, rows_a, sem_a).start()
                _gather(ja + 1, rows_b, sem_b).wait()
                _scale(rows_b, ja + 1)
                pltpu.sync_copy(rows_b, acc_sh.at[dst_v.at[ja + 1]], add=True)
                return 0
            lax.fori_loop(0, (KJ - 1) // 2, pair_body, 0)

            jt = KJ - 1
            _gather(jt, rows_a, sem_a).wait()
            _scale(rows_a, jt)
            pltpu.sync_copy(rows_a, acc_sh.at[dst_v.at[jt]], add=True)
            return 0
        lax.fori_loop(0, NG, group_body, 0)

        plsc.subcore_barrier()
        pltpu.sync_copy(acc_sh.at[pl.ds(r0, REXT)],
                        out_hbm.at[c, pl.ds(r0, REXT)])

    return pl.kernel(
        body,
        out_type=jax.ShapeDtypeStruct((2, N, D), jnp.float32),
        mesh=mesh,
        compiler_params=pltpu.CompilerParams(use_tc_tiling_on_sc=False),
        scratch_types=[
            pltpu.VMEM_SHARED((N, D), jnp.float32),
            pltpu.VMEM((KJ, CHUNK), jnp.int32),
            pltpu.VMEM((KJ, CHUNK), jnp.int32),
            pltpu.VMEM((KJ, CHUNK), jnp.float32),
            pltpu.VMEM((CHUNK, D), jnp.float32),
            pltpu.VMEM((CHUNK, D), jnp.float32),
            pltpu.SemaphoreType.DMA,
            pltpu.SemaphoreType.DMA,
        ],
    )


def kernel(x, edge_index, edge_weight, W1a, W1s, b1, W2a, W2s, b2,
           Wd, bd, Wx1, bx1, Wx2, bx2, Wx3, bx3, Wpi, bpi):
    srcg = edge_index[0].reshape(NW * NG, KJ, CHUNK)
    dstg = edge_index[1].reshape(NW * NG, KJ, CHUNK)
    wg = edge_weight.reshape(NW * NG, KJ, CHUNK)
    W2a_p = jnp.pad(W2a, ((0, 0), (0, LP - L)))
    W2s_p = jnp.pad(W2s, ((0, 0), (0, LP - L)))
    b2_p = jnp.pad(b2, (0, LP - L))
    Wd_p = jnp.pad(Wd, ((0, LP - L), (0, 0)))
    Wx1_p = jnp.pad(Wx1, ((0, LP - L), (0, 0)))

    t1 = _matmul(x, W1a)
    s1p = _make_sc_spmm(IN)(t1, srcg, dstg, wg)
    t2, u2 = _stage2(s1p, x, W1s, b1, W2a_p, W2s_p, b2_p)
    s2p = _make_sc_spmm(LP)(t2, srcg, dstg, wg)
    hA, P = _decoder(s2p, u2, Wd_p, bd, Wx1_p, bx1, Wx2, bx2, Wx3, bx3,
                     Wpi, bpi)
    A_out = _bilinear(hA)
    return (A_out, P)
